# baseline (speedup 1.0000x reference)
import functools

import jax
import jax.numpy as jnp
from jax import lax
from jax.experimental import pallas as pl
from jax.experimental.pallas import tpu as pltpu
from jax.experimental.pallas import tpu_sc as plsc

_CHUNK = 128
_NBUF = 5


def _emb_lookup(ids_flat, table_pad, n_steps, nc, nw):
    Dp = table_pad.shape[1]
    N = ids_flat.shape[0]
    n_per_w = n_steps * _CHUNK
    mesh = plsc.VectorSubcoreMesh(core_axis_name="c", subcore_axis_name="s")

    @functools.partial(
        pl.kernel,
        mesh=mesh,
        out_type=jax.ShapeDtypeStruct((N, Dp), jnp.float32),
        compiler_params=pltpu.CompilerParams(use_tc_tiling_on_sc=False),
        scratch_types=[
            pltpu.VMEM((n_per_w,), jnp.int32),
            pltpu.VMEM((_NBUF, _CHUNK, Dp), jnp.float32),
            pltpu.SemaphoreType.DMA,
            pltpu.SemaphoreType.DMA,
        ],
    )
    def emb(ids_hbm, table_hbm, out_hbm, idx_v, rows_v, gsem, osem):
        wid = lax.axis_index("s") * nc + lax.axis_index("c")
        base = wid * n_per_w
        pltpu.sync_copy(ids_hbm.at[pl.ds(base, n_per_w)], idx_v)

        def gather_copy(ci, buf):
            return pltpu.make_async_copy(
                table_hbm.at[idx_v.at[pl.ds(ci * _CHUNK, _CHUNK)]],
                rows_v.at[buf],
                gsem,
            )

        def out_copy(ci, buf):
            return pltpu.make_async_copy(
                rows_v.at[buf],
                out_hbm.at[pl.ds(base + ci * _CHUNK, _CHUNK)],
                osem,
            )

        for b in range(_NBUF):
            gather_copy(b, b).start()

        def body(g):
            for i in range(_NBUF):
                j = g + i
                gather_copy(j, i).wait()
                out_copy(j, i).start()
                out_copy(j, i).wait()
                gather_copy(j + _NBUF, i).start()

        pl.loop(0, n_steps - _NBUF, step=_NBUF)(body)

        for i in range(_NBUF):
            j = n_steps - _NBUF + i
            gather_copy(j, i).wait()
            out_copy(j, i).start()
            out_copy(j, i).wait()

    return emb(ids_flat, table_pad)


def kernel(ids, table):
    B, H = ids.shape
    V, D = table.shape
    N = B * H
    info = plsc.get_sparse_core_info()
    nc, ns = info.num_cores, info.num_subcores
    nw = nc * ns
    n_steps = N // (nw * _CHUNK)
    ids_flat = ids.reshape(N).astype(jnp.int32)
    table_pad = jnp.concatenate([table, table], axis=1)
    out = _emb_lookup(ids_flat, table_pad, n_steps, nc, nw)
    return out[:, :D].reshape(B, H, D)


# speedup vs baseline: 1.4520x; 1.4520x over previous
import functools

import jax
import jax.numpy as jnp
from jax import lax
from jax.experimental import pallas as pl
from jax.experimental.pallas import tpu as pltpu
from jax.experimental.pallas import tpu_sc as plsc

_CHUNK = 128
_NBUF = 5
_VBLK = 4096


def _tc_transpose_pad(tabT):
    D, V = tabT.shape

    def body(in_ref, out_ref):
        x = in_ref[...]
        xt = jnp.swapaxes(x, 0, 1)
        out_ref[...] = jnp.concatenate([xt, xt], axis=1)

    return pl.pallas_call(
        body,
        grid=(pl.cdiv(V, _VBLK),),
        in_specs=[pl.BlockSpec((D, _VBLK), lambda i: (0, i))],
        out_specs=pl.BlockSpec((_VBLK, 2 * D), lambda i: (i, 0)),
        out_shape=jax.ShapeDtypeStruct((V, 2 * D), jnp.float32),
    )(tabT)


def _sc_gather(ids_flat, table_pad, n_steps, nc, nw):
    Dp = table_pad.shape[1]
    N = ids_flat.shape[0]
    n_per_w = n_steps * _CHUNK
    mesh = plsc.VectorSubcoreMesh(core_axis_name="c", subcore_axis_name="s")

    @functools.partial(
        pl.kernel,
        mesh=mesh,
        out_type=jax.ShapeDtypeStruct((N, Dp), jnp.float32),
        compiler_params=pltpu.CompilerParams(use_tc_tiling_on_sc=True),
        scratch_types=[
            pltpu.VMEM((n_per_w,), jnp.int32),
            pltpu.VMEM((_NBUF, _CHUNK, Dp), jnp.float32),
            pltpu.SemaphoreType.DMA,
            pltpu.SemaphoreType.DMA,
        ],
    )
    def emb(ids_hbm, table_hbm, out_hbm, idx_v, rows_v, gsem, osem):
        wid = lax.axis_index("s") * nc + lax.axis_index("c")
        base = wid * n_per_w
        pltpu.sync_copy(ids_hbm.at[pl.ds(base, n_per_w)], idx_v)

        def gather_copy(ci, buf):
            return pltpu.make_async_copy(
                table_hbm.at[idx_v.at[pl.ds(ci * _CHUNK, _CHUNK)]],
                rows_v.at[buf],
                gsem,
            )

        def out_copy(ci, buf):
            return pltpu.make_async_copy(
                rows_v.at[buf],
                out_hbm.at[pl.ds(base + ci * _CHUNK, _CHUNK)],
                osem,
            )

        for b in range(_NBUF):
            gather_copy(b, b).start()

        def body(g):
            for i in range(_NBUF):
                j = g + i
                gather_copy(j, i).wait()
                out_copy(j, i).start()
                out_copy(j, i).wait()
                gather_copy(j + _NBUF, i).start()

        pl.loop(0, n_steps - _NBUF, step=_NBUF)(body)

        for i in range(_NBUF):
            j = n_steps - _NBUF + i
            gather_copy(j, i).wait()
            out_copy(j, i).start()
            out_copy(j, i).wait()

    return emb(ids_flat, table_pad)


def kernel(ids, table):
    B, H = ids.shape
    V, D = table.shape
    N = B * H
    info = plsc.get_sparse_core_info()
    nc, ns = info.num_cores, info.num_subcores
    nw = nc * ns
    n_steps = N // (nw * _CHUNK)
    ids_flat = ids.reshape(N).astype(jnp.int32)
    tabT = jnp.swapaxes(table, 0, 1)
    table_pad = _tc_transpose_pad(tabT)
    out = _sc_gather(ids_flat, table_pad, n_steps, nc, nw)
    return out[:, :D].reshape(B, H, D)


# speedup vs baseline: 1.4530x; 1.0007x over previous
import functools

import jax
import jax.numpy as jnp
from jax import lax
from jax.experimental import pallas as pl
from jax.experimental.pallas import tpu as pltpu
from jax.experimental.pallas import tpu_sc as plsc

_CHUNK = 128
_NBUF = 5
_VBLK = 4096


def _tc_transpose_pad(tabT):
    D, V = tabT.shape

    def body(in_ref, out_ref):
        x = in_ref[...]
        eye = jnp.eye(D, dtype=jnp.float32)
        xt = jax.lax.dot_general(
            x, eye, (((0,), (0,)), ((), ())),
            preferred_element_type=jnp.float32,
        )
        out_ref[...] = jnp.concatenate([xt, xt], axis=1)

    return pl.pallas_call(
        body,
        grid=(pl.cdiv(V, _VBLK),),
        in_specs=[pl.BlockSpec((D, _VBLK), lambda i: (0, i))],
        out_specs=pl.BlockSpec((_VBLK, 2 * D), lambda i: (i, 0)),
        out_shape=jax.ShapeDtypeStruct((V, 2 * D), jnp.float32),
    )(tabT)


def _sc_gather(ids_flat, table_pad, n_steps, nc, nw):
    Dp = table_pad.shape[1]
    N = ids_flat.shape[0]
    n_per_w = n_steps * _CHUNK
    mesh = plsc.VectorSubcoreMesh(core_axis_name="c", subcore_axis_name="s")

    @functools.partial(
        pl.kernel,
        mesh=mesh,
        out_type=jax.ShapeDtypeStruct((N, Dp), jnp.float32),
        compiler_params=pltpu.CompilerParams(use_tc_tiling_on_sc=True),
        scratch_types=[
            pltpu.VMEM((n_per_w,), jnp.int32),
            pltpu.VMEM((_NBUF, _CHUNK, Dp), jnp.float32),
            pltpu.SemaphoreType.DMA,
            pltpu.SemaphoreType.DMA,
        ],
    )
    def emb(ids_hbm, table_hbm, out_hbm, idx_v, rows_v, gsem, osem):
        wid = lax.axis_index("s") * nc + lax.axis_index("c")
        base = wid * n_per_w
        pltpu.sync_copy(ids_hbm.at[pl.ds(base, n_per_w)], idx_v)

        def gather_copy(ci, buf):
            return pltpu.make_async_copy(
                table_hbm.at[idx_v.at[pl.ds(ci * _CHUNK, _CHUNK)]],
                rows_v.at[buf],
                gsem,
            )

        def out_copy(ci, buf):
            return pltpu.make_async_copy(
                rows_v.at[buf],
                out_hbm.at[pl.ds(base + ci * _CHUNK, _CHUNK)],
                osem,
            )

        for b in range(_NBUF):
            gather_copy(b, b).start()

        def body(g):
            for i in range(_NBUF):
                j = g + i
                gather_copy(j, i).wait()
                out_copy(j, i).start()
                out_copy(j, i).wait()
                gather_copy(j + _NBUF, i).start()

        pl.loop(0, n_steps - _NBUF, step=_NBUF)(body)

        for i in range(_NBUF):
            j = n_steps - _NBUF + i
            gather_copy(j, i).wait()
            out_copy(j, i).start()
            out_copy(j, i).wait()

    return emb(ids_flat, table_pad)


def kernel(ids, table):
    B, H = ids.shape
    V, D = table.shape
    N = B * H
    info = plsc.get_sparse_core_info()
    nc, ns = info.num_cores, info.num_subcores
    nw = nc * ns
    n_steps = N // (nw * _CHUNK)
    ids_flat = ids.reshape(N).astype(jnp.int32)
    tabT = jnp.swapaxes(table, 0, 1)
    table_pad = _tc_transpose_pad(tabT)
    out = _sc_gather(ids_flat, table_pad, n_steps, nc, nw)
    return out[:, :D].reshape(B, H, D)


# speedup vs baseline: 1.6334x; 1.1242x over previous
import functools

import jax
import jax.numpy as jnp
from jax import lax
from jax.experimental import pallas as pl
from jax.experimental.pallas import tpu as pltpu
from jax.experimental.pallas import tpu_sc as plsc

_CHUNK = 128
_NBUF = 5
_VBLK = 8192


def _tc_transpose_pad(tabT):
    D, V = tabT.shape

    def body(in_ref, out_ref):
        x = in_ref[...]
        xt = jnp.swapaxes(x, 0, 1)
        out_ref[...] = jnp.concatenate([xt, xt], axis=1)

    return pl.pallas_call(
        body,
        grid=(pl.cdiv(V, _VBLK),),
        in_specs=[pl.BlockSpec((D, _VBLK), lambda i: (0, i))],
        out_specs=pl.BlockSpec((_VBLK, 2 * D), lambda i: (i, 0)),
        out_shape=jax.ShapeDtypeStruct((V, 2 * D), jnp.float32),
    )(tabT)


def _sc_gather(ids_flat, table_pad, n_steps, nc, nw):
    Dp = table_pad.shape[1]
    N = ids_flat.shape[0]
    n_per_w = n_steps * _CHUNK
    mesh = plsc.VectorSubcoreMesh(core_axis_name="c", subcore_axis_name="s")

    @functools.partial(
        pl.kernel,
        mesh=mesh,
        out_type=jax.ShapeDtypeStruct((N, Dp), jnp.float32),
        compiler_params=pltpu.CompilerParams(use_tc_tiling_on_sc=True),
        scratch_types=[
            pltpu.VMEM((n_per_w,), jnp.int32),
            pltpu.VMEM((_NBUF, _CHUNK, Dp), jnp.float32),
            pltpu.SemaphoreType.DMA,
            pltpu.SemaphoreType.DMA,
        ],
    )
    def emb(ids_hbm, table_hbm, out_hbm, idx_v, rows_v, gsem, osem):
        wid = lax.axis_index("s") * nc + lax.axis_index("c")
        base = wid * n_per_w
        pltpu.sync_copy(ids_hbm.at[pl.ds(base, n_per_w)], idx_v)

        def gather_copy(ci, buf):
            return pltpu.make_async_copy(
                table_hbm.at[idx_v.at[pl.ds(ci * _CHUNK, _CHUNK)]],
                rows_v.at[buf],
                gsem,
            )

        def out_copy(ci, buf):
            return pltpu.make_async_copy(
                rows_v.at[buf],
                out_hbm.at[pl.ds(base + ci * _CHUNK, _CHUNK)],
                osem,
            )

        for b in range(_NBUF):
            gather_copy(b, b).start()

        def body(g):
            for i in range(_NBUF):
                j = g + i
                gather_copy(j, i).wait()
                out_copy(j, i).start()
                out_copy(j, i).wait()
                gather_copy(j + _NBUF, i).start()

        pl.loop(0, n_steps - _NBUF, step=_NBUF)(body)

        for i in range(_NBUF):
            j = n_steps - _NBUF + i
            gather_copy(j, i).wait()
            out_copy(j, i).start()
            out_copy(j, i).wait()

    return emb(ids_flat, table_pad)


def kernel(ids, table):
    B, H = ids.shape
    V, D = table.shape
    N = B * H
    info = plsc.get_sparse_core_info()
    nc, ns = info.num_cores, info.num_subcores
    nw = nc * ns
    n_steps = N // (nw * _CHUNK)
    ids_flat = ids.reshape(N).astype(jnp.int32)
    tabT = jnp.swapaxes(table, 0, 1)
    table_pad = _tc_transpose_pad(tabT)
    out = _sc_gather(ids_flat, table_pad, n_steps, nc, nw)
    return out[:, :D].reshape(B, H, D)


# speedup vs baseline: 1.7359x; 1.0628x over previous
import functools

import jax
import jax.numpy as jnp
from jax import lax
from jax.experimental import pallas as pl
from jax.experimental.pallas import tpu as pltpu
from jax.experimental.pallas import tpu_sc as plsc

_CHUNK = 128
_NBUF = 5
_VBLK = 16384


def _tc_transpose_pad(tabT):
    D, V = tabT.shape

    def body(in_ref, out_ref):
        x = in_ref[...]
        xt = jnp.swapaxes(x, 0, 1)
        out_ref[...] = jnp.concatenate([xt, xt], axis=1)

    return pl.pallas_call(
        body,
        grid=(pl.cdiv(V, _VBLK),),
        in_specs=[pl.BlockSpec((D, _VBLK), lambda i: (0, i))],
        out_specs=pl.BlockSpec((_VBLK, 2 * D), lambda i: (i, 0)),
        out_shape=jax.ShapeDtypeStruct((V, 2 * D), jnp.float32),
    )(tabT)


def _sc_gather(ids_flat, table_pad, n_steps, nc, nw):
    Dp = table_pad.shape[1]
    N = ids_flat.shape[0]
    n_per_w = n_steps * _CHUNK
    mesh = plsc.VectorSubcoreMesh(core_axis_name="c", subcore_axis_name="s")

    @functools.partial(
        pl.kernel,
        mesh=mesh,
        out_type=jax.ShapeDtypeStruct((N, Dp), jnp.float32),
        compiler_params=pltpu.CompilerParams(use_tc_tiling_on_sc=True),
        scratch_types=[
            pltpu.VMEM((n_per_w,), jnp.int32),
            pltpu.VMEM((_NBUF, _CHUNK, Dp), jnp.float32),
            pltpu.SemaphoreType.DMA,
            pltpu.SemaphoreType.DMA,
        ],
    )
    def emb(ids_hbm, table_hbm, out_hbm, idx_v, rows_v, gsem, osem):
        wid = lax.axis_index("s") * nc + lax.axis_index("c")
        base = wid * n_per_w
        pltpu.sync_copy(ids_hbm.at[pl.ds(base, n_per_w)], idx_v)

        def gather_copy(ci, buf):
            return pltpu.make_async_copy(
                table_hbm.at[idx_v.at[pl.ds(ci * _CHUNK, _CHUNK)]],
                rows_v.at[buf],
                gsem,
            )

        def out_copy(ci, buf):
            return pltpu.make_async_copy(
                rows_v.at[buf],
                out_hbm.at[pl.ds(base + ci * _CHUNK, _CHUNK)],
                osem,
            )

        for b in range(_NBUF):
            gather_copy(b, b).start()

        def body(g):
            for i in range(_NBUF):
                j = g + i
                gather_copy(j, i).wait()
                out_copy(j, i).start()
                out_copy(j, i).wait()
                gather_copy(j + _NBUF, i).start()

        pl.loop(0, n_steps - _NBUF, step=_NBUF)(body)

        for i in range(_NBUF):
            j = n_steps - _NBUF + i
            gather_copy(j, i).wait()
            out_copy(j, i).start()
            out_copy(j, i).wait()

    return emb(ids_flat, table_pad)


def kernel(ids, table):
    B, H = ids.shape
    V, D = table.shape
    N = B * H
    info = plsc.get_sparse_core_info()
    nc, ns = info.num_cores, info.num_subcores
    nw = nc * ns
    n_steps = N // (nw * _CHUNK)
    ids_flat = ids.reshape(N).astype(jnp.int32)
    tabT = jnp.swapaxes(table, 0, 1)
    table_pad = _tc_transpose_pad(tabT)
    out = _sc_gather(ids_flat, table_pad, n_steps, nc, nw)
    return out[:, :D].reshape(B, H, D)


# speedup vs baseline: 1.7721x; 1.0208x over previous
import functools

import jax
import jax.numpy as jnp
from jax import lax
from jax.experimental import pallas as pl
from jax.experimental.pallas import tpu as pltpu
from jax.experimental.pallas import tpu_sc as plsc

_CHUNK = 128
_NBUF = 5
_VBLK = 24576


def _tc_transpose_pad(tabT):
    D, V = tabT.shape

    def body(in_ref, out_ref):
        x = in_ref[...]
        xt = jnp.swapaxes(x, 0, 1)
        out_ref[...] = jnp.concatenate([xt, xt], axis=1)

    return pl.pallas_call(
        body,
        grid=(pl.cdiv(V, _VBLK),),
        in_specs=[pl.BlockSpec((D, _VBLK), lambda i: (0, i))],
        out_specs=pl.BlockSpec((_VBLK, 2 * D), lambda i: (i, 0)),
        out_shape=jax.ShapeDtypeStruct((V, 2 * D), jnp.float32),
    )(tabT)


def _sc_gather(ids_flat, table_pad, n_steps, nc, nw):
    Dp = table_pad.shape[1]
    N = ids_flat.shape[0]
    n_per_w = n_steps * _CHUNK
    mesh = plsc.VectorSubcoreMesh(core_axis_name="c", subcore_axis_name="s")

    @functools.partial(
        pl.kernel,
        mesh=mesh,
        out_type=jax.ShapeDtypeStruct((N, Dp), jnp.float32),
        compiler_params=pltpu.CompilerParams(use_tc_tiling_on_sc=True),
        scratch_types=[
            pltpu.VMEM((n_per_w,), jnp.int32),
            pltpu.VMEM((_NBUF, _CHUNK, Dp), jnp.float32),
            pltpu.SemaphoreType.DMA,
            pltpu.SemaphoreType.DMA,
        ],
    )
    def emb(ids_hbm, table_hbm, out_hbm, idx_v, rows_v, gsem, osem):
        wid = lax.axis_index("s") * nc + lax.axis_index("c")
        base = wid * n_per_w
        pltpu.sync_copy(ids_hbm.at[pl.ds(base, n_per_w)], idx_v)

        def gather_copy(ci, buf):
            return pltpu.make_async_copy(
                table_hbm.at[idx_v.at[pl.ds(ci * _CHUNK, _CHUNK)]],
                rows_v.at[buf],
                gsem,
            )

        def out_copy(ci, buf):
            return pltpu.make_async_copy(
                rows_v.at[buf],
                out_hbm.at[pl.ds(base + ci * _CHUNK, _CHUNK)],
                osem,
            )

        for b in range(_NBUF):
            gather_copy(b, b).start()

        def body(g):
            for i in range(_NBUF):
                j = g + i
                gather_copy(j, i).wait()
                out_copy(j, i).start()
                out_copy(j, i).wait()
                gather_copy(j + _NBUF, i).start()

        pl.loop(0, n_steps - _NBUF, step=_NBUF)(body)

        for i in range(_NBUF):
            j = n_steps - _NBUF + i
            gather_copy(j, i).wait()
            out_copy(j, i).start()
            out_copy(j, i).wait()

    return emb(ids_flat, table_pad)


def kernel(ids, table):
    B, H = ids.shape
    V, D = table.shape
    N = B * H
    info = plsc.get_sparse_core_info()
    nc, ns = info.num_cores, info.num_subcores
    nw = nc * ns
    n_steps = N // (nw * _CHUNK)
    ids_flat = ids.reshape(N).astype(jnp.int32)
    tabT = jnp.swapaxes(table, 0, 1)
    table_pad = _tc_transpose_pad(tabT)
    out = _sc_gather(ids_flat, table_pad, n_steps, nc, nw)
    return out[:, :D].reshape(B, H, D)


# speedup vs baseline: 1.7777x; 1.0031x over previous
import functools

import jax
import jax.numpy as jnp
from jax import lax
from jax.experimental import pallas as pl
from jax.experimental.pallas import tpu as pltpu
from jax.experimental.pallas import tpu_sc as plsc

_CHUNK = 128
_NBUF = 5
_VBLK = 28672


def _tc_transpose_pad(tabT):
    D, V = tabT.shape

    def body(in_ref, out_ref):
        x = in_ref[...]
        xt = jnp.swapaxes(x, 0, 1)
        out_ref[...] = jnp.concatenate([xt, xt], axis=1)

    return pl.pallas_call(
        body,
        grid=(pl.cdiv(V, _VBLK),),
        in_specs=[pl.BlockSpec((D, _VBLK), lambda i: (0, i))],
        out_specs=pl.BlockSpec((_VBLK, 2 * D), lambda i: (i, 0)),
        out_shape=jax.ShapeDtypeStruct((V, 2 * D), jnp.float32),
    )(tabT)


def _sc_gather(ids_flat, table_pad, n_steps, nc, nw):
    Dp = table_pad.shape[1]
    N = ids_flat.shape[0]
    n_per_w = n_steps * _CHUNK
    mesh = plsc.VectorSubcoreMesh(core_axis_name="c", subcore_axis_name="s")

    @functools.partial(
        pl.kernel,
        mesh=mesh,
        out_type=jax.ShapeDtypeStruct((N, Dp), jnp.float32),
        compiler_params=pltpu.CompilerParams(use_tc_tiling_on_sc=True),
        scratch_types=[
            pltpu.VMEM((n_per_w,), jnp.int32),
            pltpu.VMEM((_NBUF, _CHUNK, Dp), jnp.float32),
            pltpu.SemaphoreType.DMA,
            pltpu.SemaphoreType.DMA,
        ],
    )
    def emb(ids_hbm, table_hbm, out_hbm, idx_v, rows_v, gsem, osem):
        wid = lax.axis_index("s") * nc + lax.axis_index("c")
        base = wid * n_per_w
        pltpu.sync_copy(ids_hbm.at[pl.ds(base, n_per_w)], idx_v)

        def gather_copy(ci, buf):
            return pltpu.make_async_copy(
                table_hbm.at[idx_v.at[pl.ds(ci * _CHUNK, _CHUNK)]],
                rows_v.at[buf],
                gsem,
            )

        def out_copy(ci, buf):
            return pltpu.make_async_copy(
                rows_v.at[buf],
                out_hbm.at[pl.ds(base + ci * _CHUNK, _CHUNK)],
                osem,
            )

        for b in range(_NBUF):
            gather_copy(b, b).start()

        def body(g):
            for i in range(_NBUF):
                j = g + i
                gather_copy(j, i).wait()
                out_copy(j, i).start()
                out_copy(j, i).wait()
                gather_copy(j + _NBUF, i).start()

        pl.loop(0, n_steps - _NBUF, step=_NBUF)(body)

        for i in range(_NBUF):
            j = n_steps - _NBUF + i
            gather_copy(j, i).wait()
            out_copy(j, i).start()
            out_copy(j, i).wait()

    return emb(ids_flat, table_pad)


def kernel(ids, table):
    B, H = ids.shape
    V, D = table.shape
    N = B * H
    info = plsc.get_sparse_core_info()
    nc, ns = info.num_cores, info.num_subcores
    nw = nc * ns
    n_steps = N // (nw * _CHUNK)
    ids_flat = ids.reshape(N).astype(jnp.int32)
    tabT = jnp.swapaxes(table, 0, 1)
    table_pad = _tc_transpose_pad(tabT)
    out = _sc_gather(ids_flat, table_pad, n_steps, nc, nw)
    return out[:, :D].reshape(B, H, D)
